# Initial kernel scaffold; baseline (speedup 1.0000x reference)
import jax
import jax.numpy as jnp
from jax.experimental import pallas as pl


def kernel(state_ids, table):
    raise NotImplementedError("write your pallas kernel here")


# speedup vs baseline: 3.6469x; 3.6469x over previous
import functools

import jax
import jax.numpy as jnp
from jax import lax
from jax.experimental import pallas as pl
from jax.experimental.pallas import tpu as pltpu
from jax.experimental.pallas import tpu_sc as plsc

EMBED_DIM = 32


@functools.cache
def _make_gather(B, D):
    info = plsc.get_sparse_core_info()
    NC, NS = info.num_cores, info.num_subcores
    NW = NC * NS
    assert B % NW == 0
    per_w = B // NW
    CH = 128
    assert per_w % CH == 0
    n_ch = per_w // CH

    mesh = plsc.VectorSubcoreMesh(core_axis_name="c", subcore_axis_name="s")

    @functools.partial(
        pl.kernel,
        mesh=mesh,
        compiler_params=pltpu.CompilerParams(use_tc_tiling_on_sc=False),
        out_type=jax.ShapeDtypeStruct((B, D), jnp.float32),
        scratch_types=[
            pltpu.VMEM((CH,), jnp.int32),
            pltpu.VMEM((CH, D), jnp.float32),
            pltpu.SemaphoreType.DMA,
        ],
    )
    def gather_kernel(idx_hbm, table_hbm, out_hbm, idx_v, rows_v, sem):
        wid = lax.axis_index("s") * NC + lax.axis_index("c")
        base = wid * per_w

        def chunk(j, carry):
            start = base + j * CH
            pltpu.sync_copy(idx_hbm.at[pl.ds(start, CH)], idx_v)
            pltpu.async_copy(table_hbm.at[idx_v], rows_v, sem).wait()
            pltpu.sync_copy(rows_v, out_hbm.at[pl.ds(start, CH)])
            return carry

        lax.fori_loop(0, n_ch, chunk, 0)

    return gather_kernel


def kernel(state_ids, table):
    B0, T = state_ids.shape
    B = B0 * T
    ids = state_ids.reshape(B).astype(jnp.int32)
    out = _make_gather(B, EMBED_DIM)(ids, table)
    return out.reshape(B0, T, EMBED_DIM)


# speedup vs baseline: 4.9924x; 1.3690x over previous
import functools

import jax
import jax.numpy as jnp
from jax import lax
from jax.experimental import pallas as pl
from jax.experimental.pallas import tpu as pltpu
from jax.experimental.pallas import tpu_sc as plsc

EMBED_DIM = 32
CH = 128
STREAMS = 8
BLK = CH * STREAMS


@functools.cache
def _make_gather(B, D):
    info = plsc.get_sparse_core_info()
    NC, NS = info.num_cores, info.num_subcores
    NW = NC * NS
    assert B % (NW * 2 * BLK) == 0
    per_w = B // NW
    n_blk = per_w // BLK

    mesh = plsc.VectorSubcoreMesh(core_axis_name="c", subcore_axis_name="s")

    @functools.partial(
        pl.kernel,
        mesh=mesh,
        compiler_params=pltpu.CompilerParams(use_tc_tiling_on_sc=False),
        out_type=jax.ShapeDtypeStruct((B, D), jnp.float32),
        scratch_types=[
            pltpu.VMEM((2, BLK), jnp.int32),
            pltpu.VMEM((2, BLK, D), jnp.float32),
            pltpu.SemaphoreType.DMA,
            pltpu.SemaphoreType.DMA,
        ],
    )
    def gather_kernel(idx_hbm, table_hbm, out_hbm, idx_v, rows_v, sem0, sem1):
        wid = lax.axis_index("s") * NC + lax.axis_index("c")
        base = wid * per_w
        sems = (sem0, sem1)

        def block(g, carry):
            starts = (base + (2 * g) * BLK, base + (2 * g + 1) * BLK)
            handles = ([], [])
            for b in range(2):
                pltpu.sync_copy(
                    idx_hbm.at[pl.ds(starts[b], BLK)], idx_v.at[b]
                )
                for k in range(STREAMS):
                    handles[b].append(
                        pltpu.async_copy(
                            table_hbm.at[idx_v.at[b, pl.ds(k * CH, CH)]],
                            rows_v.at[b, pl.ds(k * CH, CH)],
                            sems[b],
                        )
                    )
            for b in range(2):
                for h in handles[b]:
                    h.wait()
                pltpu.sync_copy(
                    rows_v.at[b], out_hbm.at[pl.ds(starts[b], BLK)]
                )
            return carry

        lax.fori_loop(0, n_blk // 2, block, 0)

    return gather_kernel


def kernel(state_ids, table):
    B0, T = state_ids.shape
    B = B0 * T
    ids = state_ids.reshape(B).astype(jnp.int32)
    out = _make_gather(B, EMBED_DIM)(ids, table)
    return out.reshape(B0, T, EMBED_DIM)
